# baseline (speedup 1.0000x reference)
import functools

import jax
import jax.numpy as jnp
from jax import lax
from jax.experimental import pallas as pl
from jax.experimental.pallas import tpu as pltpu
from jax.experimental.pallas import tpu_sc as plsc

_BINS = 50
_EPS = 1e-8
_B = 32
_N = 512 * 512
_NC = 2
_NS = 16
_NW = _NC * _NS
_CH = 32768

_NSPLIT = 1
_G = _B // _NSPLIT
_WPS = _NW // _G


def _stage1_body(yp_ref, yt_ref, idx_ref, mm_ref):
    yp = yp_ref[0]
    p = jnp.maximum(jnp.sqrt(yp[0] * yp[0] + yp[1] * yp[1]), 1e-6)
    yt = yt_ref[0]
    t = jnp.maximum(jnp.sqrt(yt[0] * yt[0] + yt[1] * yt[1]), 1e-6)
    pmin = jnp.min(p)
    pmax = jnp.max(p)
    scale = _BINS / jnp.maximum(pmax - pmin, 1e-30)
    idxp = jnp.minimum(jnp.floor((p - pmin) * scale).astype(jnp.int32), _BINS - 1)
    in_t = (t >= pmin) & (t <= pmax)
    idxt = jnp.minimum(jnp.floor((t - pmin) * scale).astype(jnp.int32), _BINS - 1)
    idxt = jnp.where(in_t, idxt + 64, 127)
    e = idxp | (idxt << 6)
    idx_ref[0] = e[0:256] | (e[256:512] << 16)
    col = lax.broadcasted_iota(jnp.int32, (1, 1, 128), 2)
    mm_ref[...] = jnp.where(col == 0, pmin, jnp.where(col == 1, pmax, 0.0))


def _stage1(y_pred, y_true, base):
    return pl.pallas_call(
        _stage1_body,
        grid=(_G,),
        in_specs=[
            pl.BlockSpec((1, 2, 512, 512), lambda s: (s + base, 0, 0, 0)),
            pl.BlockSpec((1, 2, 512, 512), lambda s: (s + base, 0, 0, 0)),
        ],
        out_specs=[
            pl.BlockSpec((1, 256, 512), lambda s: (s, 0, 0)),
            pl.BlockSpec((1, 1, 128), lambda s: (s, 0, 0)),
        ],
        out_shape=[
            jax.ShapeDtypeStruct((_G, 256, 512), jnp.int32),
            jax.ShapeDtypeStruct((_G, 1, 128), jnp.float32),
        ],
    )(y_pred, y_true)


_CROWS = _CH // 512
_WROWS = 256 // _WPS
_NCHUNK = _WROWS // _CROWS


def _sc_hist_body(idx_hbm, out_hbm, buf0, buf1, hist, sem0, sem1):
    wid = lax.axis_index("s") * _NC + lax.axis_index("c")
    sample = wid // _WPS
    rowbase = (wid % _WPS) * _WROWS
    zero = jnp.zeros((16,), jnp.float32)

    def zrow(r, carry):
        hist[pl.ds(r * 16, 16)] = zero
        return carry

    lax.fori_loop(0, 128, zrow, 0)

    lane = lax.iota(jnp.int32, 16)
    ones = jnp.ones((16,), jnp.float32)
    sems = (sem0, sem1)
    bufs = (buf0, buf1)

    def src(ci):
        return idx_hbm.at[sample, pl.ds(rowbase + ci * _CROWS, _CROWS), :]

    pltpu.async_copy(src(0), bufs[0], sems[0])
    for ci in range(_NCHUNK):
        b = ci % 2
        if ci + 1 < _NCHUNK:
            pltpu.async_copy(src(ci + 1), bufs[(ci + 1) % 2], sems[(ci + 1) % 2])
        pltpu.make_async_copy(src(ci), bufs[b], sems[b]).wait()
        bref = bufs[b]

        @plsc.parallel_loop(0, _CH // 16, unroll=8)
        def ibody(j):
            v = bref[j >> 5, pl.ds((j & 31) * 16, 16)]
            e0 = v & 0xFFFF
            e1 = lax.shift_right_logical(v, 16)
            lo0 = ((e0 << 4) & 0x3F0) | lane
            hi0 = (lax.shift_right_logical(e0, 2) & 0x7F0) | lane
            lo1 = ((e1 << 4) & 0x3F0) | lane
            hi1 = (lax.shift_right_logical(e1, 2) & 0x7F0) | lane
            plsc.addupdate_scatter(hist, [lo0], ones)
            plsc.addupdate_scatter(hist, [hi0], ones)
            plsc.addupdate_scatter(hist, [lo1], ones)
            plsc.addupdate_scatter(hist, [hi1], ones)

    pltpu.sync_copy(hist, out_hbm.at[wid])


def _sc_hist(idx):
    mesh = plsc.VectorSubcoreMesh(core_axis_name="c", subcore_axis_name="s")
    f = pl.kernel(
        _sc_hist_body,
        out_type=jax.ShapeDtypeStruct((_NW, 2048), jnp.float32),
        mesh=mesh,
        compiler_params=pltpu.CompilerParams(
            needs_layout_passes=False, use_tc_tiling_on_sc=True
        ),
        scratch_types=[
            pltpu.VMEM((_CROWS, 512), jnp.int32),
            pltpu.VMEM((_CROWS, 512), jnp.int32),
            pltpu.VMEM((2048,), jnp.float32),
            pltpu.SemaphoreType.DMA,
            pltpu.SemaphoreType.DMA,
        ],
    )
    return f(idx)


def _kl_body(h_ref, mm_ref, out_ref):
    h = jnp.sum(h_ref[...], axis=(1, 3))
    cp = h[:, 0:64]
    ct = h[:, 64:128]
    mm = mm_ref[...]
    pmin = mm[:, 0:1]
    pmax = mm[:, 1:2]
    valid = lax.broadcasted_iota(jnp.int32, (_B, 64), 1) < _BINS
    cp = jnp.where(valid, cp, 0.0)
    ct = jnp.where(valid, ct, 0.0)
    tot_p = jnp.maximum(jnp.sum(cp, axis=1, keepdims=True), 1.0)
    tot_t = jnp.maximum(jnp.sum(ct, axis=1, keepdims=True), 1.0)
    w = jnp.maximum(pmax - pmin, 1e-30) / _BINS
    hp = jnp.where(valid, cp / (w * tot_p) + _EPS, 0.0)
    ht = jnp.where(valid, ct / (w * tot_t) + _EPS, 0.0)
    hp = hp / jnp.sum(hp, axis=1, keepdims=True)
    ht = ht / jnp.sum(ht, axis=1, keepdims=True)
    ratio = jnp.where(valid, ht / hp, 1.0)
    kl = jnp.sum(jnp.where(valid, ht * jnp.log(ratio), 0.0), axis=1)
    out_ref[...] = jnp.broadcast_to(jnp.sum(kl) / _B, (1, 1))


def _kl(h, mm):
    return pl.pallas_call(
        _kl_body,
        out_shape=jax.ShapeDtypeStruct((1, 1), jnp.float32),
    )(h, mm)


def kernel(y_pred, y_true):
    hists = []
    mms = []
    for g in range(_NSPLIT):
        idx, mm = _stage1(y_pred, y_true, g * _G)
        hw = _sc_hist(idx)
        hists.append(hw.reshape(_G, _WPS, 128, 16))
        mms.append(mm.reshape(_G, 128))
    h = jnp.concatenate(hists, axis=0)
    mm = jnp.concatenate(mms, axis=0)
    out = _kl(h, mm)
    return out.reshape(())


# speedup vs baseline: 1.0618x; 1.0618x over previous
import functools

import jax
import jax.numpy as jnp
from jax import lax
from jax.experimental import pallas as pl
from jax.experimental.pallas import tpu as pltpu
from jax.experimental.pallas import tpu_sc as plsc

_BINS = 50
_EPS = 1e-8
_B = 32
_N = 512 * 512
_NC = 2
_NS = 16
_NW = _NC * _NS
_CH = 32768

_NSPLIT = 2
_G = _B // _NSPLIT
_WPS = _NW // _G


def _stage1_body(yp_ref, yt_ref, idx_ref, mm_ref):
    yp = yp_ref[0]
    p = jnp.maximum(jnp.sqrt(yp[0] * yp[0] + yp[1] * yp[1]), 1e-6)
    yt = yt_ref[0]
    t = jnp.maximum(jnp.sqrt(yt[0] * yt[0] + yt[1] * yt[1]), 1e-6)
    pmin = jnp.min(p)
    pmax = jnp.max(p)
    scale = _BINS / jnp.maximum(pmax - pmin, 1e-30)
    idxp = jnp.minimum(jnp.floor((p - pmin) * scale).astype(jnp.int32), _BINS - 1)
    in_t = (t >= pmin) & (t <= pmax)
    idxt = jnp.minimum(jnp.floor((t - pmin) * scale).astype(jnp.int32), _BINS - 1)
    idxt = jnp.where(in_t, idxt + 64, 127)
    e = idxp | (idxt << 6)
    idx_ref[0] = e[0:256] | (e[256:512] << 16)
    col = lax.broadcasted_iota(jnp.int32, (1, 1, 128), 2)
    mm_ref[...] = jnp.where(col == 0, pmin, jnp.where(col == 1, pmax, 0.0))


def _stage1(y_pred, y_true, base):
    return pl.pallas_call(
        _stage1_body,
        grid=(_G,),
        in_specs=[
            pl.BlockSpec((1, 2, 512, 512), lambda s: (s + base, 0, 0, 0)),
            pl.BlockSpec((1, 2, 512, 512), lambda s: (s + base, 0, 0, 0)),
        ],
        out_specs=[
            pl.BlockSpec((1, 256, 512), lambda s: (s, 0, 0)),
            pl.BlockSpec((1, 1, 128), lambda s: (s, 0, 0)),
        ],
        out_shape=[
            jax.ShapeDtypeStruct((_G, 256, 512), jnp.int32),
            jax.ShapeDtypeStruct((_G, 1, 128), jnp.float32),
        ],
    )(y_pred, y_true)


_CROWS = _CH // 512
_WROWS = 256 // _WPS
_NCHUNK = _WROWS // _CROWS


def _sc_hist_body(idx_hbm, out_hbm, buf0, buf1, hist, sem0, sem1):
    wid = lax.axis_index("s") * _NC + lax.axis_index("c")
    sample = wid // _WPS
    rowbase = (wid % _WPS) * _WROWS
    zero = jnp.zeros((16,), jnp.float32)

    def zrow(r, carry):
        hist[pl.ds(r * 16, 16)] = zero
        return carry

    lax.fori_loop(0, 128, zrow, 0)

    lane = lax.iota(jnp.int32, 16)
    ones = jnp.ones((16,), jnp.float32)
    sems = (sem0, sem1)
    bufs = (buf0, buf1)

    def src(ci):
        return idx_hbm.at[sample, pl.ds(rowbase + ci * _CROWS, _CROWS), :]

    pltpu.async_copy(src(0), bufs[0], sems[0])
    for ci in range(_NCHUNK):
        b = ci % 2
        if ci + 1 < _NCHUNK:
            pltpu.async_copy(src(ci + 1), bufs[(ci + 1) % 2], sems[(ci + 1) % 2])
        pltpu.make_async_copy(src(ci), bufs[b], sems[b]).wait()
        bref = bufs[b]

        @plsc.parallel_loop(0, _CH // 16, unroll=8)
        def ibody(j):
            v = bref[j >> 5, pl.ds((j & 31) * 16, 16)]
            e0 = v & 0xFFFF
            e1 = lax.shift_right_logical(v, 16)
            lo0 = ((e0 << 4) & 0x3F0) | lane
            hi0 = (lax.shift_right_logical(e0, 2) & 0x7F0) | lane
            lo1 = ((e1 << 4) & 0x3F0) | lane
            hi1 = (lax.shift_right_logical(e1, 2) & 0x7F0) | lane
            plsc.addupdate_scatter(hist, [lo0], ones)
            plsc.addupdate_scatter(hist, [hi0], ones)
            plsc.addupdate_scatter(hist, [lo1], ones)
            plsc.addupdate_scatter(hist, [hi1], ones)

    pltpu.sync_copy(hist, out_hbm.at[wid])


def _sc_hist(idx):
    mesh = plsc.VectorSubcoreMesh(core_axis_name="c", subcore_axis_name="s")
    f = pl.kernel(
        _sc_hist_body,
        out_type=jax.ShapeDtypeStruct((_NW, 2048), jnp.float32),
        mesh=mesh,
        compiler_params=pltpu.CompilerParams(
            needs_layout_passes=False, use_tc_tiling_on_sc=True
        ),
        scratch_types=[
            pltpu.VMEM((_CROWS, 512), jnp.int32),
            pltpu.VMEM((_CROWS, 512), jnp.int32),
            pltpu.VMEM((2048,), jnp.float32),
            pltpu.SemaphoreType.DMA,
            pltpu.SemaphoreType.DMA,
        ],
    )
    return f(idx)


def _kl_body(h_ref, mm_ref, out_ref):
    h = jnp.sum(h_ref[...], axis=(1, 3))
    cp = h[:, 0:64]
    ct = h[:, 64:128]
    mm = mm_ref[...]
    pmin = mm[:, 0:1]
    pmax = mm[:, 1:2]
    valid = lax.broadcasted_iota(jnp.int32, (_B, 64), 1) < _BINS
    cp = jnp.where(valid, cp, 0.0)
    ct = jnp.where(valid, ct, 0.0)
    tot_p = jnp.maximum(jnp.sum(cp, axis=1, keepdims=True), 1.0)
    tot_t = jnp.maximum(jnp.sum(ct, axis=1, keepdims=True), 1.0)
    w = jnp.maximum(pmax - pmin, 1e-30) / _BINS
    hp = jnp.where(valid, cp / (w * tot_p) + _EPS, 0.0)
    ht = jnp.where(valid, ct / (w * tot_t) + _EPS, 0.0)
    hp = hp / jnp.sum(hp, axis=1, keepdims=True)
    ht = ht / jnp.sum(ht, axis=1, keepdims=True)
    ratio = jnp.where(valid, ht / hp, 1.0)
    kl = jnp.sum(jnp.where(valid, ht * jnp.log(ratio), 0.0), axis=1)
    out_ref[...] = jnp.broadcast_to(jnp.sum(kl) / _B, (1, 1))


def _kl(h, mm):
    return pl.pallas_call(
        _kl_body,
        out_shape=jax.ShapeDtypeStruct((1, 1), jnp.float32),
    )(h, mm)


def kernel(y_pred, y_true):
    hists = []
    mms = []
    for g in range(_NSPLIT):
        idx, mm = _stage1(y_pred, y_true, g * _G)
        hw = _sc_hist(idx)
        hists.append(hw.reshape(_G, _WPS, 128, 16))
        mms.append(mm.reshape(_G, 128))
    h = jnp.concatenate(hists, axis=0)
    mm = jnp.concatenate(mms, axis=0)
    out = _kl(h, mm)
    return out.reshape(())


# speedup vs baseline: 1.2047x; 1.1345x over previous
import functools

import jax
import jax.numpy as jnp
from jax import lax
from jax.experimental import pallas as pl
from jax.experimental.pallas import tpu as pltpu
from jax.experimental.pallas import tpu_sc as plsc

_BINS = 50
_EPS = 1e-8
_B = 32
_N = 512 * 512
_NC = 2
_NS = 16
_NW = _NC * _NS
_CH = 32768

_NSPLIT = 2
_G = _B // _NSPLIT
_WPS = _NW // _G


_CK = 16


def _stage1_body(yp_ref, yt_ref, idx_ref, mm_ref):
    def q_chunk(ref, r0, rows):
        a = ref[0, 0, pl.ds(r0, rows), :]
        b = ref[0, 1, pl.ds(r0, rows), :]
        return a * a + b * b

    def norm_chunk(ref, r0, rows):
        q = jnp.maximum(q_chunk(ref, r0, rows), 1e-12)
        return q * lax.rsqrt(q)

    accmin = q_chunk(yp_ref, 0, _CK)
    accmax = accmin
    for c in range(1, 512 // _CK):
        q = q_chunk(yp_ref, c * _CK, _CK)
        accmin = jnp.minimum(accmin, q)
        accmax = jnp.maximum(accmax, q)
    pmin = jnp.maximum(jnp.sqrt(jnp.min(accmin)), 1e-6)
    pmax = jnp.maximum(jnp.sqrt(jnp.max(accmax)), 1e-6)
    scale = _BINS / jnp.maximum(pmax - pmin, 1e-30)
    off = -pmin * scale

    def ebits(r0):
        p = norm_chunk(yp_ref, r0, _CK)
        ep = jnp.clip(jnp.floor(p * scale + off), 0.0, float(_BINS - 1))
        ep = ep.astype(jnp.int32)
        t = norm_chunk(yt_ref, r0, _CK)
        et = jnp.clip(jnp.floor(t * scale + off), -1.0, float(_BINS - 1))
        et = et.astype(jnp.int32) + 64
        et = jnp.where(t <= pmax, et, 127)
        return ep | (et << 6)

    for c in range(256 // _CK):
        e_top = ebits(c * _CK)
        e_bot = ebits(256 + c * _CK)
        idx_ref[0, pl.ds(c * _CK, _CK), :] = e_top | (e_bot << 16)

    col = lax.broadcasted_iota(jnp.int32, (1, 1, 128), 2)
    mm_ref[...] = jnp.where(col == 0, pmin, jnp.where(col == 1, pmax, 0.0))


def _stage1(y_pred, y_true, base):
    return pl.pallas_call(
        _stage1_body,
        grid=(_G,),
        in_specs=[
            pl.BlockSpec((1, 2, 512, 512), lambda s: (s + base, 0, 0, 0)),
            pl.BlockSpec((1, 2, 512, 512), lambda s: (s + base, 0, 0, 0)),
        ],
        out_specs=[
            pl.BlockSpec((1, 256, 512), lambda s: (s, 0, 0)),
            pl.BlockSpec((1, 1, 128), lambda s: (s, 0, 0)),
        ],
        out_shape=[
            jax.ShapeDtypeStruct((_G, 256, 512), jnp.int32),
            jax.ShapeDtypeStruct((_G, 1, 128), jnp.float32),
        ],
    )(y_pred, y_true)


_CROWS = _CH // 512
_WROWS = 256 // _WPS
_NCHUNK = _WROWS // _CROWS


def _sc_hist_body(idx_hbm, out_hbm, buf0, buf1, hist, sem0, sem1):
    wid = lax.axis_index("s") * _NC + lax.axis_index("c")
    sample = wid // _WPS
    rowbase = (wid % _WPS) * _WROWS
    zero = jnp.zeros((16,), jnp.float32)

    def zrow(r, carry):
        hist[pl.ds(r * 16, 16)] = zero
        return carry

    lax.fori_loop(0, 128, zrow, 0)

    lane = lax.iota(jnp.int32, 16)
    ones = jnp.ones((16,), jnp.float32)
    sems = (sem0, sem1)
    bufs = (buf0, buf1)

    def src(ci):
        return idx_hbm.at[sample, pl.ds(rowbase + ci * _CROWS, _CROWS), :]

    pltpu.async_copy(src(0), bufs[0], sems[0])
    for ci in range(_NCHUNK):
        b = ci % 2
        if ci + 1 < _NCHUNK:
            pltpu.async_copy(src(ci + 1), bufs[(ci + 1) % 2], sems[(ci + 1) % 2])
        pltpu.make_async_copy(src(ci), bufs[b], sems[b]).wait()
        bref = bufs[b]

        @plsc.parallel_loop(0, _CH // 16, unroll=8)
        def ibody(j):
            v = bref[j >> 5, pl.ds((j & 31) * 16, 16)]
            e0 = v & 0xFFFF
            e1 = lax.shift_right_logical(v, 16)
            lo0 = ((e0 << 4) & 0x3F0) | lane
            hi0 = (lax.shift_right_logical(e0, 2) & 0x7F0) | lane
            lo1 = ((e1 << 4) & 0x3F0) | lane
            hi1 = (lax.shift_right_logical(e1, 2) & 0x7F0) | lane
            plsc.addupdate_scatter(hist, [lo0], ones)
            plsc.addupdate_scatter(hist, [hi0], ones)
            plsc.addupdate_scatter(hist, [lo1], ones)
            plsc.addupdate_scatter(hist, [hi1], ones)

    pltpu.sync_copy(hist, out_hbm.at[wid])


def _sc_hist(idx):
    mesh = plsc.VectorSubcoreMesh(core_axis_name="c", subcore_axis_name="s")
    f = pl.kernel(
        _sc_hist_body,
        out_type=jax.ShapeDtypeStruct((_NW, 2048), jnp.float32),
        mesh=mesh,
        compiler_params=pltpu.CompilerParams(
            needs_layout_passes=False, use_tc_tiling_on_sc=True
        ),
        scratch_types=[
            pltpu.VMEM((_CROWS, 512), jnp.int32),
            pltpu.VMEM((_CROWS, 512), jnp.int32),
            pltpu.VMEM((2048,), jnp.float32),
            pltpu.SemaphoreType.DMA,
            pltpu.SemaphoreType.DMA,
        ],
    )
    return f(idx)


def _kl_body(h_ref, mm_ref, out_ref):
    h = jnp.sum(h_ref[...], axis=(1, 3))
    cp = h[:, 0:64]
    ct = h[:, 64:128]
    mm = mm_ref[...]
    pmin = mm[:, 0:1]
    pmax = mm[:, 1:2]
    valid = lax.broadcasted_iota(jnp.int32, (_B, 64), 1) < _BINS
    cp = jnp.where(valid, cp, 0.0)
    ct = jnp.where(valid, ct, 0.0)
    tot_p = jnp.maximum(jnp.sum(cp, axis=1, keepdims=True), 1.0)
    tot_t = jnp.maximum(jnp.sum(ct, axis=1, keepdims=True), 1.0)
    w = jnp.maximum(pmax - pmin, 1e-30) / _BINS
    hp = jnp.where(valid, cp / (w * tot_p) + _EPS, 0.0)
    ht = jnp.where(valid, ct / (w * tot_t) + _EPS, 0.0)
    hp = hp / jnp.sum(hp, axis=1, keepdims=True)
    ht = ht / jnp.sum(ht, axis=1, keepdims=True)
    ratio = jnp.where(valid, ht / hp, 1.0)
    kl = jnp.sum(jnp.where(valid, ht * jnp.log(ratio), 0.0), axis=1)
    out_ref[...] = jnp.broadcast_to(jnp.sum(kl) / _B, (1, 1))


def _kl(h, mm):
    return pl.pallas_call(
        _kl_body,
        out_shape=jax.ShapeDtypeStruct((1, 1), jnp.float32),
    )(h, mm)


def kernel(y_pred, y_true):
    hists = []
    mms = []
    for g in range(_NSPLIT):
        idx, mm = _stage1(y_pred, y_true, g * _G)
        hw = _sc_hist(idx)
        hists.append(hw.reshape(_G, _WPS, 128, 16))
        mms.append(mm.reshape(_G, 128))
    h = jnp.concatenate(hists, axis=0)
    mm = jnp.concatenate(mms, axis=0)
    out = _kl(h, mm)
    return out.reshape(())
